# baseline (speedup 1.0000x reference)
import functools

import jax
import jax.numpy as jnp
from jax import lax
from jax.experimental import pallas as pl
from jax.experimental.pallas import tpu as pltpu
from jax.experimental.pallas import tpu_sc as plsc

_VOCAB = 37
_PAD_W = 128
_NUM_CORES = 2
_NUM_SUBCORES = 16
_NW = _NUM_CORES * _NUM_SUBCORES
_DEPTH = 4
_LANES = 16
_IDX_STRIDE = 56


def _logits_table_body(table_ref, w_ref, b_ref, p_ref):
    p = lax.dot_general(
        table_ref[...], w_ref[...], (((1,), (1,)), ((), ())),
        preferred_element_type=jnp.float32)
    p = p + b_ref[...]
    p_ref[...] = jnp.concatenate(
        [p, jnp.zeros((_VOCAB, _PAD_W - _VOCAB), jnp.float32)], axis=-1)


def _build_logits_table(table, W, b):
    return pl.pallas_call(
        _logits_table_body,
        out_shape=jax.ShapeDtypeStruct((_VOCAB, _PAD_W), jnp.float32),
    )(table, W, b.reshape(1, _VOCAB))


def _make_sc_gather(bsz, seqlen):
    b_per_tile = bsz // _NW
    assert b_per_tile * _NW == bsz
    assert seqlen <= _IDX_STRIDE and seqlen <= 128
    tok_per_tile = b_per_tile * seqlen
    assert tok_per_tile % 8 == 0
    mesh = plsc.VectorSubcoreMesh(
        core_axis_name="c", subcore_axis_name="s")

    scratch = [
        pltpu.VMEM((tok_per_tile + 2 * _LANES,), jnp.int32),
        pltpu.VMEM((b_per_tile * _IDX_STRIDE,), jnp.int32),
    ]
    scratch += [
        pltpu.VMEM((seqlen, _PAD_W), jnp.float32) for _ in range(_DEPTH)
    ]
    scratch += [
        pltpu.VMEM((seqlen, _VOCAB), jnp.float32) for _ in range(_DEPTH)
    ]
    scratch += [pltpu.SemaphoreType.DMA for _ in range(2 * _DEPTH)]

    @functools.partial(
        pl.kernel,
        out_type=jax.ShapeDtypeStruct((bsz, seqlen, _VOCAB), jnp.float32),
        mesh=mesh,
        scratch_types=scratch,
        compiler_params=pltpu.CompilerParams(needs_layout_passes=False),
    )
    def sc_gather(p_hbm, idx_hbm, out_hbm, idx_v, idxp_v, *rest):
        bufs = rest[:_DEPTH]
        obufs = rest[_DEPTH:2 * _DEPTH]
        gsems = rest[2 * _DEPTH:3 * _DEPTH]
        ssems = rest[3 * _DEPTH:]
        wid = lax.axis_index("s") * _NUM_CORES + lax.axis_index("c")
        b0 = wid * b_per_tile

        pltpu.sync_copy(
            idx_hbm.at[pl.ds(b0 * seqlen, tok_per_tile)],
            idx_v.at[pl.ds(0, tok_per_tile)])

        n_vec = (seqlen + _LANES - 1) // _LANES

        def repack(i, carry):
            src = i * seqlen
            dst = i * _IDX_STRIDE
            for k in range(n_vec):
                idxp_v[pl.ds(dst + k * _LANES, _LANES)] = (
                    idx_v[pl.ds(src + k * _LANES, _LANES)])
            return carry

        lax.fori_loop(0, b_per_tile, repack, 0)

        offs = (0, _LANES, _VOCAB - _LANES)

        def repack_rows(buf, obuf):
            def row_fn(i, carry):
                for o in offs:
                    obuf[i, pl.ds(o, _LANES)] = buf[i, pl.ds(o, _LANES)]
                return carry
            lax.fori_loop(0, seqlen, row_fn, 0)

        gcopies = [None] * _DEPTH
        scopies = [None] * _DEPTH
        for it in range(b_per_tile + 1):
            if it < b_per_tile:
                sl = it % _DEPTH
                if scopies[sl] is not None:
                    scopies[sl].wait()
                gcopies[sl] = pltpu.async_copy(
                    p_hbm.at[idxp_v.at[pl.ds(it * _IDX_STRIDE, seqlen)]],
                    bufs[sl], gsems[sl])
            it2 = it - 1
            if it2 >= 0:
                sl2 = it2 % _DEPTH
                gcopies[sl2].wait()
                repack_rows(bufs[sl2], obufs[sl2])
                scopies[sl2] = pltpu.async_copy(
                    obufs[sl2], out_hbm.at[b0 + it2], ssems[sl2])
        for it2 in range(max(0, b_per_tile - _DEPTH), b_per_tile):
            scopies[it2 % _DEPTH].wait()

    return sc_gather


def kernel(indices, table, W, b):
    bsz, seqlen = indices.shape
    p = _build_logits_table(table, W, b)
    idx_flat = indices.reshape(-1)
    return _make_sc_gather(bsz, seqlen)(p, idx_flat)


# speedup vs baseline: 5.1490x; 5.1490x over previous
import functools

import jax
import jax.numpy as jnp
from jax import lax
from jax.experimental import pallas as pl
from jax.experimental.pallas import tpu as pltpu
from jax.experimental.pallas import tpu_sc as plsc

_VOCAB = 37
_NUM_CORES = 2
_NUM_SUBCORES = 16
_NW = _NUM_CORES * _NUM_SUBCORES
_LANES = 16
_BW = 128
_NL = 10
_DEPTH = 2


def _logits_table_body(table_ref, w_ref, b_ref, pt_ref):
    pt = lax.dot_general(
        w_ref[...], table_ref[...], (((1,), (1,)), ((), ())),
        preferred_element_type=jnp.float32)
    pt_ref[...] = pt + b_ref[...]


def _build_logits_table(table, W, b):
    return pl.pallas_call(
        _logits_table_body,
        out_shape=jax.ShapeDtypeStruct((_VOCAB, _VOCAB), jnp.float32),
    )(table, W, b.reshape(_VOCAB, 1))


def _make_sc_lookup(bsz, seqlen):
    assert bsz % (_NW * _BW) == 0 and bsz == _NW * _BW
    n_chunks = (seqlen + _NL - 1) // _NL
    assert seqlen % _NL == 0
    mesh = plsc.VectorSubcoreMesh(
        core_axis_name="c", subcore_axis_name="s")

    scratch = [
        pltpu.VMEM((seqlen, _BW), jnp.int32),
        pltpu.VMEM((_VOCAB, _VOCAB), jnp.float32),
    ]
    scratch += [
        pltpu.VMEM((_NL * _VOCAB, _BW), jnp.float32) for _ in range(_DEPTH)
    ]
    scratch += [pltpu.SemaphoreType.DMA for _ in range(_DEPTH)]

    @functools.partial(
        pl.kernel,
        out_type=jax.ShapeDtypeStruct((seqlen, _VOCAB, bsz), jnp.float32),
        mesh=mesh,
        scratch_types=scratch,
        compiler_params=pltpu.CompilerParams(needs_layout_passes=False),
    )
    def sc_lookup(pt_hbm, idxt_hbm, out_hbm, idx_v, pt_v, *rest):
        stages = rest[:_DEPTH]
        ssems = rest[_DEPTH:]
        wid = lax.axis_index("s") * _NUM_CORES + lax.axis_index("c")
        b0 = wid * _BW

        pltpu.sync_copy(idxt_hbm.at[:, pl.ds(b0, _BW)], idx_v)
        pltpu.sync_copy(pt_hbm, pt_v)

        n_groups = _BW // _LANES
        vrows = [jnp.full((_LANES,), v, jnp.int32) for v in range(_VOCAB)]

        def make_rows(stage, l0):
            def row_fn(li, carry):
                toks = [idx_v[l0 + li, pl.ds(g * _LANES, _LANES)]
                        for g in range(n_groups)]
                for v in range(_VOCAB):
                    for g in range(n_groups):
                        stage[li * _VOCAB + v, pl.ds(g * _LANES, _LANES)] = (
                            plsc.load_gather(pt_v, [vrows[v], toks[g]]))
                return carry
            lax.fori_loop(0, _NL, row_fn, 0)

        scopies = [None] * _DEPTH
        for c in range(n_chunks):
            sl = c % _DEPTH
            if scopies[sl] is not None:
                for cp in scopies[sl]:
                    cp.wait()
            make_rows(stages[sl], c * _NL)
            scopies[sl] = [
                pltpu.async_copy(
                    stages[sl].at[pl.ds(li * _VOCAB, _VOCAB), :],
                    out_hbm.at[c * _NL + li, :, pl.ds(b0, _BW)],
                    ssems[sl])
                for li in range(_NL)
            ]
        for c in range(max(0, n_chunks - _DEPTH), n_chunks):
            for cp in scopies[c % _DEPTH]:
                cp.wait()

    return sc_lookup


def kernel(indices, table, W, b):
    bsz, seqlen = indices.shape
    pt = _build_logits_table(table, W, b)
    idx_t = indices.T
    out = _make_sc_lookup(bsz, seqlen)(pt, idx_t)
    return out.transpose(2, 0, 1)


# speedup vs baseline: 5.4447x; 1.0574x over previous
import functools

import jax
import jax.numpy as jnp
from jax import lax
from jax.experimental import pallas as pl
from jax.experimental.pallas import tpu as pltpu
from jax.experimental.pallas import tpu_sc as plsc

_VOCAB = 37
_NUM_CORES = 2
_NUM_SUBCORES = 16
_NW = _NUM_CORES * _NUM_SUBCORES
_LANES = 16
_REP = _VOCAB * _LANES
_BW = 128
_NL = 10
_DEPTH = 2


def _logits_table_body(table_ref, w_ref, b_ref, pt_ref):
    pt = lax.dot_general(
        w_ref[...], table_ref[...], (((1,), (1,)), ((), ())),
        preferred_element_type=jnp.float32)
    pt = pt + b_ref[...]
    expand = jnp.asarray(
        lax.broadcasted_iota(jnp.int32, (_VOCAB, _REP), 0)
        == lax.broadcasted_iota(jnp.int32, (_VOCAB, _REP), 1) // _LANES,
        jnp.float32)
    pt_ref[...] = lax.dot_general(
        pt, expand, (((1,), (0,)), ((), ())),
        preferred_element_type=jnp.float32)


def _build_logits_table(table, W, b):
    return pl.pallas_call(
        _logits_table_body,
        out_shape=jax.ShapeDtypeStruct((_VOCAB, _REP), jnp.float32),
    )(table, W, b.reshape(_VOCAB, 1))


def _make_sc_lookup(bsz, seqlen):
    assert bsz % (_NW * _BW) == 0 and bsz == _NW * _BW
    n_chunks = (seqlen + _NL - 1) // _NL
    assert seqlen % _NL == 0
    mesh = plsc.VectorSubcoreMesh(
        core_axis_name="c", subcore_axis_name="s")

    scratch = [
        pltpu.VMEM((seqlen, _BW), jnp.int32),
        pltpu.VMEM((_VOCAB, _REP), jnp.float32),
    ]
    scratch += [
        pltpu.VMEM((_NL * _VOCAB, _BW), jnp.float32) for _ in range(_DEPTH)
    ]
    scratch += [pltpu.SemaphoreType.DMA for _ in range(_DEPTH)]

    @functools.partial(
        pl.kernel,
        out_type=jax.ShapeDtypeStruct((seqlen, _VOCAB, bsz), jnp.float32),
        mesh=mesh,
        scratch_types=scratch,
        compiler_params=pltpu.CompilerParams(needs_layout_passes=False),
    )
    def sc_lookup(pt_hbm, idxt_hbm, out_hbm, idx_v, pt_v, *rest):
        stages = rest[:_DEPTH]
        ssems = rest[_DEPTH:]
        wid = lax.axis_index("s") * _NUM_CORES + lax.axis_index("c")
        b0 = wid * _BW

        pltpu.sync_copy(idxt_hbm.at[:, pl.ds(b0, _BW)], idx_v)
        pltpu.sync_copy(pt_hbm, pt_v)

        n_groups = _BW // _LANES
        lane = lax.iota(jnp.int32, _LANES)
        vrows = [jnp.full((_LANES,), v, jnp.int32) for v in range(_VOCAB)]

        def make_rows(stage, l0):
            def row_fn(li, carry):
                tcols = [
                    idx_v[l0 + li, pl.ds(g * _LANES, _LANES)] * _LANES + lane
                    for g in range(n_groups)
                ]
                for v in range(_VOCAB):
                    for g in range(n_groups):
                        stage[li * _VOCAB + v, pl.ds(g * _LANES, _LANES)] = (
                            plsc.load_gather(pt_v, [vrows[v], tcols[g]]))
                return carry
            lax.fori_loop(0, _NL, row_fn, 0)

        scopies = [None] * _DEPTH
        for c in range(n_chunks):
            sl = c % _DEPTH
            if scopies[sl] is not None:
                for cp in scopies[sl]:
                    cp.wait()
            make_rows(stages[sl], c * _NL)
            scopies[sl] = [
                pltpu.async_copy(
                    stages[sl].at[pl.ds(li * _VOCAB, _VOCAB), :],
                    out_hbm.at[c * _NL + li, :, pl.ds(b0, _BW)],
                    ssems[sl])
                for li in range(_NL)
            ]
        for c in range(max(0, n_chunks - _DEPTH), n_chunks):
            for cp in scopies[c % _DEPTH]:
                cp.wait()

    return sc_lookup


def kernel(indices, table, W, b):
    bsz, seqlen = indices.shape
    pt = _build_logits_table(table, W, b)
    idx_t = indices.T
    out = _make_sc_lookup(bsz, seqlen)(pt, idx_t)
    return out.transpose(2, 0, 1)


# speedup vs baseline: 7.4438x; 1.3671x over previous
import functools

import jax
import jax.numpy as jnp
from jax import lax
from jax.experimental import pallas as pl
from jax.experimental.pallas import tpu as pltpu
from jax.experimental.pallas import tpu_sc as plsc

_VOCAB = 37
_NUM_CORES = 2
_NUM_SUBCORES = 16
_NW = _NUM_CORES * _NUM_SUBCORES
_LANES = 16
_REP = _VOCAB * _LANES
_BW = 128
_NL = 10
_DEPTH = 2


def _logits_table_body(table_ref, w_ref, b_ref, pt_ref):
    pt = lax.dot_general(
        w_ref[...], table_ref[...], (((1,), (1,)), ((), ())),
        preferred_element_type=jnp.float32)
    pt_ref[...] = pt + b_ref[...]


def _build_logits_table(table, W, b):
    return pl.pallas_call(
        _logits_table_body,
        out_shape=jax.ShapeDtypeStruct((_VOCAB, _VOCAB), jnp.float32),
    )(table, W, b.reshape(_VOCAB, 1))


def _make_sc_lookup(bsz, seqlen):
    assert bsz % (_NW * _BW) == 0 and bsz == _NW * _BW
    n_chunks = (seqlen + _NL - 1) // _NL
    assert seqlen % _NL == 0
    mesh = plsc.VectorSubcoreMesh(
        core_axis_name="c", subcore_axis_name="s")

    scratch = [
        pltpu.VMEM((seqlen, _BW), jnp.int32),
        pltpu.VMEM((_VOCAB, _VOCAB), jnp.float32),
        pltpu.VMEM((_DEPTH * _NL * _VOCAB, _BW), jnp.float32),
        pltpu.SemaphoreType.DMA,
    ]

    @functools.partial(
        pl.kernel,
        out_type=jax.ShapeDtypeStruct((seqlen, _VOCAB, bsz), jnp.float32),
        mesh=mesh,
        scratch_types=scratch,
        compiler_params=pltpu.CompilerParams(needs_layout_passes=False),
    )
    def sc_lookup(pt_hbm, idxt_hbm, out_hbm, idx_v, pt_v, stage, ssem):
        wid = lax.axis_index("s") * _NUM_CORES + lax.axis_index("c")
        b0 = wid * _BW

        pltpu.sync_copy(idxt_hbm.at[:, pl.ds(b0, _BW)], idx_v)
        pltpu.sync_copy(pt_hbm, pt_v)

        n_groups = _BW // _LANES
        rows_per_chunk = _NL * _VOCAB
        vrows = [jnp.full((_LANES,), v, jnp.int32) for v in range(_VOCAB)]

        def wait_chunk():
            for _ in range(_NL):
                pltpu.make_async_copy(
                    stage.at[pl.ds(0, _VOCAB), :],
                    out_hbm.at[0, :, pl.ds(b0, _BW)],
                    ssem).wait()

        def chunk_fn(c, carry):
            base = (c % _DEPTH) * rows_per_chunk
            l0 = c * _NL

            @pl.when(c >= _DEPTH)
            def _():
                wait_chunk()

            @plsc.parallel_loop(0, _NL)
            def row_fn(li):
                tcols = [
                    idx_v[l0 + li, pl.ds(g * _LANES, _LANES)]
                    for g in range(n_groups)
                ]
                for v in range(_VOCAB):
                    for g in range(n_groups):
                        stage[base + li * _VOCAB + v,
                              pl.ds(g * _LANES, _LANES)] = (
                            plsc.load_gather(pt_v, [vrows[v], tcols[g]]))

            for li in range(_NL):
                pltpu.async_copy(
                    stage.at[pl.ds(base + li * _VOCAB, _VOCAB), :],
                    out_hbm.at[l0 + li, :, pl.ds(b0, _BW)],
                    ssem)
            return carry

        lax.fori_loop(0, n_chunks, chunk_fn, 0)
        for _ in range(min(_DEPTH, n_chunks)):
            wait_chunk()

    return sc_lookup


def kernel(indices, table, W, b):
    bsz, seqlen = indices.shape
    pt = _build_logits_table(table, W, b)
    idx_t = indices.T
    out = _make_sc_lookup(bsz, seqlen)(pt, idx_t)
    return out.transpose(2, 0, 1)
